# baseline (speedup 1.0000x reference)
import jax
import jax.numpy as jnp
from jax import lax
from jax.experimental import pallas as pl
from jax.experimental.pallas import tpu as pltpu
from jax.experimental.pallas import tpu_sc as plsc

_B = 32
_TT = 4096
_H = 128
_NC = 2
_NS = 16
_NW = _NC * _NS
_CH = _TT // _NW
_L = 16
_ROWS = 1 + _TT
_NBUF = 4
_NCHUNK = 32
_RPC = _CH // _NCHUNK
_G = _H // _L


def _embed_body(tokens_t, table, pos, marker, out, pos_v, idx_v, mk_v,
                mkblk_v, emb_v, sem_g, sem_o, sem_m):
    w = lax.axis_index("s") * _NC + lax.axis_index("c")
    t0 = w * _CH
    out0 = pl.multiple_of(_B + w * (_CH * _B), 8)

    pltpu.sync_copy(tokens_t.at[pl.ds(w * _NCHUNK, _NCHUNK), :], idx_v)
    pltpu.sync_copy(pos.at[pl.ds(t0, _CH), :], pos_v)

    @pl.when(w == _NW - 1)
    def _markers():
        pltpu.sync_copy(marker, mk_v)

        @plsc.parallel_loop(0, _B)
        def _fill(i):
            for g in range(_G):
                s = pl.ds(g * _L, _L)
                mkblk_v[i, s] = mk_v[s]
        pltpu.async_copy(mkblk_v, out.at[pl.ds(0, _B), :], sem_m).wait()

    def start_gather(c, buf):
        pltpu.async_copy(table.at[idx_v.at[c]], emb_v.at[buf],
                         sem_g.at[buf])

    def wait_gather(c, buf):
        pltpu.make_async_copy(table.at[idx_v.at[c]], emb_v.at[buf],
                              sem_g.at[buf]).wait()

    def start_store(c, buf):
        dst = out.at[pl.ds(pl.multiple_of(out0 + c * (_RPC * _B), 8),
                           _RPC * _B), :]
        pltpu.async_copy(emb_v.at[buf], dst, sem_o.at[buf])

    def wait_store(c, buf):
        dst = out.at[pl.ds(pl.multiple_of(out0 + c * (_RPC * _B), 8),
                           _RPC * _B), :]
        pltpu.make_async_copy(emb_v.at[buf], dst, sem_o.at[buf]).wait()

    def add_chunk(c, buf):
        @plsc.parallel_loop(0, _RPC * _B)
        def _add(row):
            r = c * _RPC + (row >> 5)
            for g in range(_G):
                s = pl.ds(g * _L, _L)
                emb_v[buf, row, s] = emb_v[buf, row, s] + pos_v[r, s]

    start_gather(0, 0)
    start_gather(1, 1)
    for c in (0, 1):
        start_gather(c + 2, (c + 2) % _NBUF)
        wait_gather(c, c % _NBUF)
        add_chunk(c, c % _NBUF)
        start_store(c, c % _NBUF)

    @pl.loop(0, (_NCHUNK - _NBUF) // _NBUF)
    def _trip(t):
        for k in range(_NBUF):
            c = 4 * t + 2 + k
            buf = (2 + k) % _NBUF
            wait_store(c - 2, k)
            start_gather(c + 2, k)
            wait_gather(c, buf)
            add_chunk(c, buf)
            start_store(c, buf)

    for c in (_NCHUNK - 2, _NCHUNK - 1):
        wait_gather(c, c % _NBUF)
        add_chunk(c, c % _NBUF)
        start_store(c, c % _NBUF)

    for c in range(_NCHUNK - _NBUF, _NCHUNK):
        wait_store(c, c % _NBUF)


@jax.jit
def _embed(tokens_t, table, pos, marker):
    kern = pl.kernel(
        _embed_body,
        out_type=jax.ShapeDtypeStruct((_ROWS * _B, _H), jnp.float32),
        mesh=plsc.VectorSubcoreMesh(
            core_axis_name="c", subcore_axis_name="s",
            num_cores=_NC, num_subcores=_NS),
        scratch_types=[
            pltpu.VMEM((_CH, _H), jnp.float32),
            pltpu.VMEM((_NCHUNK, _RPC * _B), jnp.int32),
            pltpu.VMEM((_H,), jnp.float32),
            pltpu.VMEM((_B, _H), jnp.float32),
            pltpu.VMEM((_NBUF, _RPC * _B, _H), jnp.float32),
            pltpu.SemaphoreType.DMA((_NBUF,)),
            pltpu.SemaphoreType.DMA((_NBUF,)),
            pltpu.SemaphoreType.DMA,
        ],
    )
    return kern(tokens_t, table, pos, marker)


def kernel(text_tokens, token_embedding, text_pos, text_marker, sep):
    del sep
    tokens_t = text_tokens.astype(jnp.int32).T.reshape(_NW * _NCHUNK,
                                                       _RPC * _B)
    res = _embed(tokens_t, token_embedding, text_pos, text_marker)
    return res.reshape(_ROWS, _B, _H).transpose(1, 0, 2)


# speedup vs baseline: 1.0156x; 1.0156x over previous
import jax
import jax.numpy as jnp
from jax import lax
from jax.experimental import pallas as pl
from jax.experimental.pallas import tpu as pltpu
from jax.experimental.pallas import tpu_sc as plsc

_B = 32
_TT = 4096
_H = 128
_NC = 2
_NS = 16
_NW = _NC * _NS
_CH = _TT // _NW
_L = 16
_ROWS = 1 + _TT
_NBUF = 4
_NCHUNK = 32
_RPC = _CH // _NCHUNK
_G = _H // _L


def _embed_body(tokens_t, table, pos, marker, out, pos_v, idx_v, mk_v,
                mkblk_v, emb_v, sem_g, sem_o, sem_m):
    w = lax.axis_index("s") * _NC + lax.axis_index("c")
    t0 = w * _CH
    out0 = pl.multiple_of(_B + w * (_CH * _B), 8)

    pltpu.sync_copy(tokens_t.at[pl.ds(w * _NCHUNK, _NCHUNK), :], idx_v)

    def start_gather(c, buf):
        pltpu.async_copy(table.at[idx_v.at[c]], emb_v.at[buf],
                         sem_g.at[buf])

    def wait_gather(c, buf):
        pltpu.make_async_copy(table.at[idx_v.at[c]], emb_v.at[buf],
                              sem_g.at[buf]).wait()

    def start_store(c, buf):
        dst = out.at[pl.ds(pl.multiple_of(out0 + c * (_RPC * _B), 8),
                           _RPC * _B), :]
        pltpu.async_copy(emb_v.at[buf], dst, sem_o.at[buf])

    def wait_store(c, buf):
        dst = out.at[pl.ds(pl.multiple_of(out0 + c * (_RPC * _B), 8),
                           _RPC * _B), :]
        pltpu.make_async_copy(emb_v.at[buf], dst, sem_o.at[buf]).wait()

    def add_chunk(c, buf):
        for rr in range(_RPC):
            r = c * _RPC + rr
            pvals = [pos_v[r, pl.ds(g * _L, _L)] for g in range(_G)]

            @plsc.parallel_loop(0, _B, unroll=1)
            def _add(b):
                row = rr * _B + b
                for g in range(_G):
                    s = pl.ds(g * _L, _L)
                    emb_v[buf, row, s] = emb_v[buf, row, s] + pvals[g]

    start_gather(0, 0)
    start_gather(1, 1)
    pltpu.sync_copy(pos.at[pl.ds(t0, _CH), :], pos_v)

    @pl.when(w == _NW - 1)
    def _markers():
        pltpu.sync_copy(marker, mk_v)
        for i in range(_B):
            for g in range(_G):
                s = pl.ds(g * _L, _L)
                mkblk_v[i, s] = mk_v[s]
        pltpu.async_copy(mkblk_v, out.at[pl.ds(0, _B), :], sem_m).wait()

    for c in (0, 1):
        start_gather(c + 2, (c + 2) % _NBUF)
        wait_gather(c, c % _NBUF)
        add_chunk(c, c % _NBUF)
        start_store(c, c % _NBUF)

    @pl.loop(0, (_NCHUNK - _NBUF) // _NBUF)
    def _trip(t):
        for k in range(_NBUF):
            c = 4 * t + 2 + k
            buf = (2 + k) % _NBUF
            wait_store(c - 2, k)
            start_gather(c + 2, k)
            wait_gather(c, buf)
            add_chunk(c, buf)
            start_store(c, buf)

    for c in (_NCHUNK - 2, _NCHUNK - 1):
        wait_gather(c, c % _NBUF)
        add_chunk(c, c % _NBUF)
        start_store(c, c % _NBUF)

    for c in range(_NCHUNK - _NBUF, _NCHUNK):
        wait_store(c, c % _NBUF)


@jax.jit
def _embed(tokens_t, table, pos, marker):
    kern = pl.kernel(
        _embed_body,
        out_type=jax.ShapeDtypeStruct((_ROWS * _B, _H), jnp.float32),
        mesh=plsc.VectorSubcoreMesh(
            core_axis_name="c", subcore_axis_name="s",
            num_cores=_NC, num_subcores=_NS),
        scratch_types=[
            pltpu.VMEM((_CH, _H), jnp.float32),
            pltpu.VMEM((_NCHUNK, _RPC * _B), jnp.int32),
            pltpu.VMEM((_H,), jnp.float32),
            pltpu.VMEM((_B, _H), jnp.float32),
            pltpu.VMEM((_NBUF, _RPC * _B, _H), jnp.float32),
            pltpu.SemaphoreType.DMA((_NBUF,)),
            pltpu.SemaphoreType.DMA((_NBUF,)),
            pltpu.SemaphoreType.DMA,
        ],
    )
    return kern(tokens_t, table, pos, marker)


def kernel(text_tokens, token_embedding, text_pos, text_marker, sep):
    del sep
    tokens_t = text_tokens.astype(jnp.int32).T.reshape(_NW * _NCHUNK,
                                                       _RPC * _B)
    res = _embed(tokens_t, token_embedding, text_pos, text_marker)
    return res.reshape(_ROWS, _B, _H).transpose(1, 0, 2)
